# Initial kernel scaffold; baseline (speedup 1.0000x reference)
import jax
import jax.numpy as jnp
from jax.experimental import pallas as pl


def kernel(x, table):
    raise NotImplementedError("write your pallas kernel here")


# speedup vs baseline: 2.0194x; 2.0194x over previous
import functools
import numpy as np
import jax
import jax.numpy as jnp
from jax import lax
from jax.experimental import pallas as pl
from jax.experimental.pallas import tpu as pltpu
from jax.experimental.pallas import tpu_sc as plsc

VOCAB = 100000
D_MODEL = 64
BATCH = 1024
SEQ_LEN = 200

_NC = 2
_NS = 16
_NW = _NC * _NS
_BPW = BATCH // _NW
_LANES = 16
_KD = D_MODEL // _LANES


def _positional_encoding(length, depth):
    half = depth / 2
    positions = np.arange(length)[:, np.newaxis]
    depths = np.arange(half)[np.newaxis, :] / half
    angle_rates = 1 / 10000 ** depths
    angle_rads = positions * angle_rates
    pos = np.concatenate([np.sin(angle_rads), np.cos(angle_rads)], axis=-1)
    return pos.astype(np.float32)


def _sc_body(table_hbm, x_hbm, pos_hbm, out_hbm, idx_v, pos_v, rows_v, gsem):
    wid = lax.axis_index("s") * _NC + lax.axis_index("c")
    row_base = wid * _BPW

    pltpu.sync_copy(pos_hbm, pos_v)

    def chunk_body(c, carry):
        pltpu.sync_copy(
            x_hbm.at[pl.ds((row_base + c) * SEQ_LEN, SEQ_LEN)], idx_v
        )
        pltpu.async_copy(table_hbm.at[idx_v], rows_v, gsem).wait()

        def r_body(r, carry2):
            for k in range(_KD):
                sl = pl.ds(k * _LANES, _LANES)
                rows_v[r, sl] = rows_v[r, sl] * 8.0 + pos_v[r, sl]
            return carry2

        lax.fori_loop(0, SEQ_LEN, r_body, 0, unroll=2)
        pltpu.sync_copy(
            rows_v, out_hbm.at[pl.ds((row_base + c) * SEQ_LEN, SEQ_LEN)]
        )
        return carry

    lax.fori_loop(0, _BPW, chunk_body, 0)


@jax.jit
def _pos_embed(x_flat, table, pos):
    mesh = plsc.VectorSubcoreMesh(
        core_axis_name="c", subcore_axis_name="s", num_cores=_NC
    )
    k = pl.kernel(
        _sc_body,
        out_type=jax.ShapeDtypeStruct((BATCH * SEQ_LEN, D_MODEL), jnp.float32),
        mesh=mesh,
        scratch_types=[
            pltpu.VMEM((SEQ_LEN,), jnp.int32),
            pltpu.VMEM((SEQ_LEN, D_MODEL), jnp.float32),
            pltpu.VMEM((SEQ_LEN, D_MODEL), jnp.float32),
            pltpu.SemaphoreType.DMA,
        ],
        compiler_params=pltpu.CompilerParams(use_tc_tiling_on_sc=False),
    )
    return k(table, x_flat, pos)


def kernel(x, table):
    pos = jnp.asarray(_positional_encoding(SEQ_LEN, D_MODEL))
    x_flat = jnp.reshape(x.astype(jnp.int32), (BATCH * SEQ_LEN,))
    out = _pos_embed(x_flat, table, pos)
    return jnp.reshape(out, (BATCH, SEQ_LEN, D_MODEL))


# speedup vs baseline: 2.8037x; 1.3883x over previous
import functools
import numpy as np
import jax
import jax.numpy as jnp
from jax import lax
from jax.experimental import pallas as pl
from jax.experimental.pallas import tpu as pltpu
from jax.experimental.pallas import tpu_sc as plsc

VOCAB = 100000
D_MODEL = 64
BATCH = 1024
SEQ_LEN = 200

_NC = 2
_NS = 16
_NW = _NC * _NS
_BPW = BATCH // _NW
_LANES = 16
_KD = D_MODEL // _LANES
_NBUF = 4


def _positional_encoding(length, depth):
    half = depth / 2
    positions = np.arange(length)[:, np.newaxis]
    depths = np.arange(half)[np.newaxis, :] / half
    angle_rates = 1 / 10000 ** depths
    angle_rads = positions * angle_rates
    pos = np.concatenate([np.sin(angle_rads), np.cos(angle_rads)], axis=-1)
    return pos.astype(np.float32)


def _sc_body(table_hbm, x_hbm, pos_hbm, out_hbm,
             idx_v, pos_v, rows_v, gsems, ssems):
    wid = lax.axis_index("s") * _NC + lax.axis_index("c")
    base = wid * _BPW * SEQ_LEN

    pltpu.sync_copy(x_hbm.at[pl.ds(base, _BPW * SEQ_LEN)], idx_v)
    pltpu.sync_copy(pos_hbm, pos_v)

    def gather(c, b):
        return pltpu.make_async_copy(
            table_hbm.at[idx_v.at[pl.ds(c * SEQ_LEN, SEQ_LEN)]],
            rows_v.at[b],
            gsems[b],
        )

    def store(c, b):
        return pltpu.make_async_copy(
            rows_v.at[b],
            out_hbm.at[pl.ds(base + c * SEQ_LEN, SEQ_LEN)],
            ssems[b],
        )

    def compute(b):
        def r_body(r, carry):
            for k in range(_KD):
                sl = pl.ds(k * _LANES, _LANES)
                rows_v[b, r, sl] = rows_v[b, r, sl] * 8.0 + pos_v[r, sl]
            return carry

        lax.fori_loop(0, SEQ_LEN, r_body, 0, unroll=4)

    for b in range(_NBUF - 1):
        gather(b, b).start()

    def outer(c4, carry):
        for b in range(_NBUF):
            c = c4 * _NBUF + b
            gather(c, b).wait()
            compute(b)
            store(c, b).start()
            bp = (b - 1) % _NBUF

            @pl.when(c >= 1)
            def _():
                store(0, bp).wait()

            @pl.when(c + _NBUF - 1 < _BPW)
            def _():
                gather(c + _NBUF - 1, bp).start()

        return carry

    lax.fori_loop(0, _BPW // _NBUF, outer, 0)
    store(0, _NBUF - 1).wait()


@jax.jit
def _pos_embed(x_flat, table, pos):
    mesh = plsc.VectorSubcoreMesh(
        core_axis_name="c", subcore_axis_name="s", num_cores=_NC
    )
    k = pl.kernel(
        _sc_body,
        out_type=jax.ShapeDtypeStruct((BATCH * SEQ_LEN, D_MODEL), jnp.float32),
        mesh=mesh,
        scratch_types=[
            pltpu.VMEM((_BPW * SEQ_LEN,), jnp.int32),
            pltpu.VMEM((SEQ_LEN, D_MODEL), jnp.float32),
            pltpu.VMEM((_NBUF, SEQ_LEN, D_MODEL), jnp.float32),
            [pltpu.SemaphoreType.DMA] * _NBUF,
            [pltpu.SemaphoreType.DMA] * _NBUF,
        ],
        compiler_params=pltpu.CompilerParams(use_tc_tiling_on_sc=False),
    )
    return k(table, x_flat, pos)


def kernel(x, table):
    pos = jnp.asarray(_positional_encoding(SEQ_LEN, D_MODEL))
    x_flat = jnp.reshape(x.astype(jnp.int32), (BATCH * SEQ_LEN,))
    out = _pos_embed(x_flat, table, pos)
    return jnp.reshape(out, (BATCH, SEQ_LEN, D_MODEL))
